# Initial kernel scaffold; baseline (speedup 1.0000x reference)
import jax
import jax.numpy as jnp
from jax.experimental import pallas as pl


def kernel(x, temp_edge_index, temp_edge_weight, edge_index, edge_weights, Wz_c, bz_c, Wr_c, br_c, Wh_c, bh_c, Wz, bz, Wr, br, Wh, bh, att, Wout, bout):
    raise NotImplementedError("write your pallas kernel here")


# speedup vs baseline: 6.7632x; 6.7632x over previous
import functools

import jax
import jax.numpy as jnp
from jax import lax
from jax.experimental import pallas as pl
from jax.experimental.pallas import tpu as pltpu
from jax.experimental.pallas import tpu_sc as plsc

B = 28
N = 207
FIN = 2
T = 36
HID = 256
E = 1656
OUT = 36

NPAD = 208
EPAD = 1664

EROWS = EPAD // 128


@functools.cache
def _make_build_adj():
    mesh = plsc.VectorSubcoreMesh(core_axis_name="c", subcore_axis_name="s")
    return pl.kernel(
        _build_adj_body,
        out_type=(
            jax.ShapeDtypeStruct((2, NPAD * NPAD), jnp.float32),
            jax.ShapeDtypeStruct((2, NPAD), jnp.float32),
        ),
        mesh=mesh,
        scratch_types=[
            pltpu.VMEM((EROWS, 128), jnp.int32),
            pltpu.VMEM((EROWS, 128), jnp.int32),
            pltpu.VMEM((EROWS, 128), jnp.float32),
            pltpu.VMEM((EROWS, 128), jnp.int32),
            pltpu.VMEM((NPAD * NPAD,), jnp.float32),
            pltpu.VMEM((NPAD,), jnp.float32),
            pltpu.VMEM_SHARED((NPAD * NPAD,), jnp.float32),
            pltpu.VMEM_SHARED((NPAD,), jnp.float32),
        ],
    )


def _build_adj(srcs, dsts, ews):
    return _make_build_adj()(srcs, dsts, ews)


def _build_adj_body(src_hbm, dst_hbm, ew_hbm, s_hbm, deg_hbm,
                    src_v, dst_v, ew_v, fidx_v, zero_v, degz_v, s_sh, deg_sh):
    wid = lax.axis_index("s") * 2 + lax.axis_index("c")

    @pl.when(wid < 2)
    def _():
        g = wid
        pltpu.sync_copy(src_hbm.at[g], src_v)
        pltpu.sync_copy(dst_hbm.at[g], dst_v)
        pltpu.sync_copy(ew_hbm.at[g], ew_v)

        zeros16 = jnp.zeros((16,), jnp.float32)

        def zero_s(i, carry):
            zero_v[pl.ds(i * 16, 16)] = zeros16
            return carry

        lax.fori_loop(0, (NPAD * NPAD) // 16, zero_s, 0)
        pltpu.sync_copy(zero_v, s_sh)

        def zero_deg(i, carry):
            degz_v[pl.ds(i * 16, 16)] = zeros16
            return carry

        lax.fori_loop(0, NPAD // 16, zero_deg, 0)
        pltpu.sync_copy(degz_v, deg_sh)

        for j in range(EROWS):
            for k in range(8):
                s16 = src_v[j, pl.ds(k * 16, 16)]
                d16 = dst_v[j, pl.ds(k * 16, 16)]
                fidx_v[j, pl.ds(k * 16, 16)] = d16 * NPAD + s16

        for j in range(EROWS):
            pltpu.sync_copy(ew_v.at[j], s_sh.at[fidx_v.at[j]], add=True)
            pltpu.sync_copy(ew_v.at[j], deg_sh.at[dst_v.at[j]], add=True)

        pltpu.sync_copy(s_sh, s_hbm.at[g])
        pltpu.sync_copy(deg_sh, deg_hbm.at[g])


def _tc_body(att_ref, xt_ref, s_ref, degr_ref, degc_ref,
             wzc_ref, wz_ref, bzc_ref, bz_ref,
             whc_ref, wh_ref, bhc_ref, bh_ref,
             wout_ref, bout_ref, o_ref):
    f32 = jnp.float32

    wz_top = wz_ref[:HID, :]
    wh_top = wh_ref[:HID, :]
    mz = jnp.dot(wzc_ref[...], wz_top, preferred_element_type=f32)
    mh = jnp.dot(whc_ref[...], wh_top, preferred_element_type=f32)
    bzv = jnp.dot(bzc_ref[...], wz_top, preferred_element_type=f32) + bz_ref[...]
    bhv = jnp.dot(bhc_ref[...], wh_top, preferred_element_type=f32) + bh_ref[...]

    a = att_ref[...]
    a = a - jnp.max(a, axis=1, keepdims=True)
    ea = jnp.exp(a)
    probs = ea / jnp.sum(ea, axis=1, keepdims=True)

    rid = lax.broadcasted_iota(jnp.int32, (NPAD, NPAD), 0)
    cid = lax.broadcasted_iota(jnp.int32, (NPAD, NPAD), 1)

    mz0, mz1 = mz[0:1, :], mz[1:2, :]
    mh0, mh1 = mh[0:1, :], mh[1:2, :]

    hs = []
    for g in range(2):
        degc = degc_ref[g] + 1.0
        degr = degr_ref[g] + 1.0
        dinv_c = lax.rsqrt(degc)
        dinv_r = lax.rsqrt(degr)
        s = s_ref[g] * dinv_c * dinv_r
        s = s + jnp.where(rid == cid, 1.0 / degc, jnp.zeros((), f32))

        a0 = jnp.dot(s, xt_ref[0, 0], preferred_element_type=f32)
        a1 = jnp.dot(s, xt_ref[0, 1], preferred_element_type=f32)

        acc = jnp.zeros((NPAD, HID), f32)
        for t in range(T):
            c0 = a0[:, t:t + 1]
            c1 = a1[:, t:t + 1]
            zl = c0 * mz0 + c1 * mz1 + bzv
            hl = c0 * mh0 + c1 * mh1 + bhv
            gt = (1.0 - jax.nn.sigmoid(zl)) * jnp.tanh(hl)
            acc = acc + probs[0:1, t:t + 1] * gt
        hs.append(jnp.maximum(acc, 0.0))

    out = (jnp.dot(hs[0], wout_ref[:HID, :], preferred_element_type=f32)
           + jnp.dot(hs[1], wout_ref[HID:, :], preferred_element_type=f32)
           + bout_ref[...])
    o_ref[0] = out


def _pad_e(v):
    return jnp.pad(v, (0, EPAD - E)).reshape(EROWS, 128)


def kernel(x, temp_edge_index, temp_edge_weight, edge_index, edge_weights,
           Wz_c, bz_c, Wr_c, br_c, Wh_c, bh_c,
           Wz, bz, Wr, br, Wh, bh, att, Wout, bout):
    srcs = jnp.stack([_pad_e(temp_edge_index[0]), _pad_e(edge_index[0])])
    dsts = jnp.stack([_pad_e(temp_edge_index[1]), _pad_e(edge_index[1])])
    ews = jnp.stack([_pad_e(temp_edge_weight), _pad_e(edge_weights)])

    s_raw, deg = _build_adj(srcs, dsts, ews)
    s_raw = s_raw.reshape(2, NPAD, NPAD)
    degr = deg.reshape(2, 1, NPAD)
    degc = deg.reshape(2, NPAD, 1)

    xt = jnp.pad(jnp.transpose(x, (0, 2, 1, 3)),
                 ((0, 0), (0, 0), (0, NPAD - N), (0, 0)))

    const = lambda *zeros: (lambda b: zeros)
    out = pl.pallas_call(
        _tc_body,
        grid=(B,),
        in_specs=[
            pl.BlockSpec((1, T), const(0, 0)),
            pl.BlockSpec((1, FIN, NPAD, T), lambda b: (b, 0, 0, 0)),
            pl.BlockSpec((2, NPAD, NPAD), const(0, 0, 0)),
            pl.BlockSpec((2, 1, NPAD), const(0, 0, 0)),
            pl.BlockSpec((2, NPAD, 1), const(0, 0, 0)),
            pl.BlockSpec((FIN, HID), const(0, 0)),
            pl.BlockSpec((2 * HID, HID), const(0, 0)),
            pl.BlockSpec((1, HID), const(0, 0)),
            pl.BlockSpec((1, HID), const(0, 0)),
            pl.BlockSpec((FIN, HID), const(0, 0)),
            pl.BlockSpec((2 * HID, HID), const(0, 0)),
            pl.BlockSpec((1, HID), const(0, 0)),
            pl.BlockSpec((1, HID), const(0, 0)),
            pl.BlockSpec((2 * HID, OUT), const(0, 0)),
            pl.BlockSpec((1, OUT), const(0, 0)),
        ],
        out_specs=pl.BlockSpec((1, NPAD, OUT), lambda b: (b, 0, 0)),
        out_shape=jax.ShapeDtypeStruct((B, NPAD, OUT), jnp.float32),
    )(att.reshape(1, T), xt, s_raw, degr, degc,
      Wz_c, Wz, bz_c.reshape(1, HID), bz.reshape(1, HID),
      Wh_c, Wh, bh_c.reshape(1, HID), bh.reshape(1, HID),
      Wout, bout.reshape(1, OUT))
    return out[:, :N, :]


# speedup vs baseline: 9.6085x; 1.4207x over previous
import functools

import jax
import jax.numpy as jnp
from jax import lax
from jax.experimental import pallas as pl
from jax.experimental.pallas import tpu as pltpu
from jax.experimental.pallas import tpu_sc as plsc

B = 28
N = 207
FIN = 2
T = 36
HID = 256
E = 1656
OUT = 36

NPAD = 208
MPAD = 216
EPAD = 1664
EROWS = EPAD // 128


@functools.cache
def _make_build_adj():
    mesh = plsc.VectorSubcoreMesh(core_axis_name="c", subcore_axis_name="s")
    return pl.kernel(
        _build_adj_body,
        out_type=(
            jax.ShapeDtypeStruct((2, NPAD * NPAD), jnp.float32),
            jax.ShapeDtypeStruct((2, NPAD), jnp.float32),
        ),
        mesh=mesh,
        scratch_types=[
            pltpu.VMEM((EROWS, 128), jnp.int32),
            pltpu.VMEM((EROWS, 128), jnp.int32),
            pltpu.VMEM((EROWS, 128), jnp.float32),
            pltpu.VMEM((EROWS, 128), jnp.int32),
            pltpu.VMEM_SHARED((NPAD * NPAD,), jnp.float32),
            pltpu.VMEM_SHARED((NPAD,), jnp.float32),
        ],
    )


def _build_adj(srcs, dsts, ews, zeros_flat, zeros_deg):
    return _make_build_adj()(srcs, dsts, ews, zeros_flat, zeros_deg)


def _build_adj_body(src_hbm, dst_hbm, ew_hbm, z_hbm, zd_hbm, s_hbm, deg_hbm,
                    src_v, dst_v, ew_v, fidx_v, s_sh, deg_sh):
    wid = lax.axis_index("s") * 2 + lax.axis_index("c")

    @pl.when(wid < 2)
    def _():
        g = wid
        pltpu.sync_copy(src_hbm.at[g], src_v)
        pltpu.sync_copy(dst_hbm.at[g], dst_v)
        pltpu.sync_copy(ew_hbm.at[g], ew_v)
        pltpu.sync_copy(z_hbm, s_sh)
        pltpu.sync_copy(zd_hbm, deg_sh)

        for j in range(EROWS):
            for k in range(8):
                s16 = src_v[j, pl.ds(k * 16, 16)]
                d16 = dst_v[j, pl.ds(k * 16, 16)]
                fidx_v[j, pl.ds(k * 16, 16)] = d16 * NPAD + s16

        for j in range(EROWS):
            pltpu.sync_copy(ew_v.at[j], s_sh.at[fidx_v.at[j]], add=True)
            pltpu.sync_copy(ew_v.at[j], deg_sh.at[dst_v.at[j]], add=True)

        pltpu.sync_copy(s_sh, s_hbm.at[g])
        pltpu.sync_copy(deg_sh, deg_hbm.at[g])


def _tc_body(att_ref, x_ref, s_ref, degr_ref, degc_ref,
             wzc_ref, wz_ref, bzc_ref, bz_ref,
             whc_ref, wh_ref, bhc_ref, bh_ref,
             wout_ref, bout_ref, o_ref):
    f32 = jnp.float32

    wz_top = wz_ref[:HID, :]
    wh_top = wh_ref[:HID, :]
    mz = jnp.dot(wzc_ref[...], wz_top, preferred_element_type=f32)
    mh = jnp.dot(whc_ref[...], wh_top, preferred_element_type=f32)
    bzv = jnp.dot(bzc_ref[...], wz_top, preferred_element_type=f32) + bz_ref[...]
    bhv = jnp.dot(bhc_ref[...], wh_top, preferred_element_type=f32) + bh_ref[...]
    maug = jnp.concatenate(
        [jnp.concatenate([-0.5 * mz, mh], axis=1),
         jnp.concatenate([-0.5 * bzv, bhv], axis=1)], axis=0)

    a = att_ref[...]
    a = a - jnp.max(a, axis=1, keepdims=True)
    ea = jnp.exp(a)
    probs = ea / jnp.sum(ea, axis=1, keepdims=True)

    rid = lax.broadcasted_iota(jnp.int32, (NPAD, NPAD), 0)
    cid = lax.broadcasted_iota(jnp.int32, (NPAD, NPAD), 1)
    aug_cid = lax.broadcasted_iota(jnp.int32, (NPAD, MPAD - NPAD), 1)
    ones_col = jnp.where(aug_cid == 0, jnp.ones((), f32),
                         jnp.zeros((), f32))

    hs = []
    for g in range(2):
        degc = degc_ref[g] + 1.0
        degr = degr_ref[g] + 1.0
        dinv_c = lax.rsqrt(degc)
        dinv_r = lax.rsqrt(degr)
        s = s_ref[g] * dinv_c * dinv_r
        s = s + jnp.where(rid == cid, 1.0 / degc, jnp.zeros((), f32))
        saug = jnp.concatenate([s, ones_col], axis=1)

        a_all = jnp.dot(saug, x_ref[0], preferred_element_type=f32)

        acc = jnp.zeros((NPAD, HID), f32)
        for t in range(T):
            zh = jnp.dot(a_all[:, 3 * t:3 * t + 3], maug,
                         preferred_element_type=f32)
            tz = jnp.tanh(zh[:, :HID])
            th = jnp.tanh(zh[:, HID:])
            gt = (0.5 + 0.5 * tz) * th
            acc = acc + probs[0:1, t:t + 1] * gt
        hs.append(jnp.maximum(acc, 0.0))

    out = (jnp.dot(hs[0], wout_ref[:HID, :], preferred_element_type=f32)
           + jnp.dot(hs[1], wout_ref[HID:, :], preferred_element_type=f32)
           + bout_ref[...])
    o_ref[0] = out


def _pad_e(v):
    return jnp.pad(v, (0, EPAD - E)).reshape(EROWS, 128)


def kernel(x, temp_edge_index, temp_edge_weight, edge_index, edge_weights,
           Wz_c, bz_c, Wr_c, br_c, Wh_c, bh_c,
           Wz, bz, Wr, br, Wh, bh, att, Wout, bout):
    srcs = jnp.stack([_pad_e(temp_edge_index[0]), _pad_e(edge_index[0])])
    dsts = jnp.stack([_pad_e(temp_edge_index[1]), _pad_e(edge_index[1])])
    ews = jnp.stack([_pad_e(temp_edge_weight), _pad_e(edge_weights)])

    s_raw, deg = _build_adj(srcs, dsts, ews,
                            jnp.zeros((NPAD * NPAD,), jnp.float32),
                            jnp.zeros((NPAD,), jnp.float32))
    s_raw = s_raw.reshape(2, NPAD, NPAD)
    degr = deg.reshape(2, 1, NPAD)
    degc = deg.reshape(2, NPAD, 1)

    xt = jnp.pad(jnp.transpose(x, (0, 1, 3, 2)),
                 ((0, 0), (0, MPAD - N), (0, 0), (0, 1)))
    xaug = xt.reshape(B, MPAD, 3 * T)
    sel = (jnp.arange(3 * T) % 3 == 2).astype(jnp.float32)
    xaug = xaug.at[:, NPAD, :].set(sel[None, :])

    const = lambda *zeros: (lambda b: zeros)
    out = pl.pallas_call(
        _tc_body,
        grid=(B,),
        in_specs=[
            pl.BlockSpec((1, T), const(0, 0)),
            pl.BlockSpec((1, MPAD, 3 * T), lambda b: (b, 0, 0)),
            pl.BlockSpec((2, NPAD, NPAD), const(0, 0, 0)),
            pl.BlockSpec((2, 1, NPAD), const(0, 0, 0)),
            pl.BlockSpec((2, NPAD, 1), const(0, 0, 0)),
            pl.BlockSpec((FIN, HID), const(0, 0)),
            pl.BlockSpec((2 * HID, HID), const(0, 0)),
            pl.BlockSpec((1, HID), const(0, 0)),
            pl.BlockSpec((1, HID), const(0, 0)),
            pl.BlockSpec((FIN, HID), const(0, 0)),
            pl.BlockSpec((2 * HID, HID), const(0, 0)),
            pl.BlockSpec((1, HID), const(0, 0)),
            pl.BlockSpec((1, HID), const(0, 0)),
            pl.BlockSpec((2 * HID, OUT), const(0, 0)),
            pl.BlockSpec((1, OUT), const(0, 0)),
        ],
        out_specs=pl.BlockSpec((1, NPAD, OUT), lambda b: (b, 0, 0)),
        out_shape=jax.ShapeDtypeStruct((B, NPAD, OUT), jnp.float32),
    )(att.reshape(1, T), xaug, s_raw, degr, degc,
      Wz_c, Wz, bz_c.reshape(1, HID), bz.reshape(1, HID),
      Wh_c, Wh, bh_c.reshape(1, HID), bh.reshape(1, HID),
      Wout, bout.reshape(1, OUT))
    return out[:, :N, :]
